# baseline (speedup 1.0000x reference)
import functools

import jax
import jax.numpy as jnp
from jax import lax
from jax.experimental import pallas as pl
from jax.experimental.pallas import tpu as pltpu
from jax.experimental.pallas import tpu_sc as plsc

EMBED_DIM = 64
BATCH = 16384
_NC, _NS = 2, 16
_NW = _NC * _NS
_BPW = BATCH // _NW
_CHUNK = 128
_NCHUNK = _BPW // _CHUNK

_mesh = plsc.VectorSubcoreMesh(core_axis_name="c", subcore_axis_name="s")


@functools.partial(
    pl.kernel,
    mesh=_mesh,
    out_type=jax.ShapeDtypeStruct((BATCH, EMBED_DIM), jnp.float32),
    scratch_types=[
        pltpu.VMEM((_NCHUNK, _CHUNK), jnp.int32),
        pltpu.VMEM((_BPW, EMBED_DIM), jnp.float32),
        pltpu.SemaphoreType.DMA,
    ],
    compiler_params=pltpu.CompilerParams(use_tc_tiling_on_sc=False),
)
def _gather_kernel(idx_hbm, table_hbm, out_hbm, idx_v, rows_v, sem):
    wid = lax.axis_index("s") * _NC + lax.axis_index("c")
    base = wid * _BPW
    pltpu.sync_copy(idx_hbm.at[wid], idx_v)
    copies = []
    for j in range(_NCHUNK):
        copies.append(
            pltpu.async_copy(
                table_hbm.at[idx_v.at[j]],
                rows_v.at[pl.ds(j * _CHUNK, _CHUNK)],
                sem,
            )
        )
    for c in copies:
        c.wait()
    pltpu.sync_copy(rows_v, out_hbm.at[pl.ds(base, _BPW)])


def kernel(class_labels, embedding_table):
    idx = class_labels.astype(jnp.int32).reshape(_NW, _NCHUNK, _CHUNK)
    out = _gather_kernel(idx, embedding_table)
    return out.reshape(BATCH, 1, EMBED_DIM)


# speedup vs baseline: 2.3644x; 2.3644x over previous
import functools

import jax
import jax.numpy as jnp
from jax import lax
from jax.experimental import pallas as pl
from jax.experimental.pallas import tpu as pltpu
from jax.experimental.pallas import tpu_sc as plsc

EMBED_DIM = 64
BATCH = 16384
_NC, _NS = 2, 16
_NW = _NC * _NS
_PAIRS = _NW // 2
_SPP = BATCH // _PAIRS
_DPW = EMBED_DIM // 2
_B = 8
_NBURST = _SPP // _B

_mesh = plsc.VectorSubcoreMesh(core_axis_name="c", subcore_axis_name="s")


@functools.partial(
    pl.kernel,
    mesh=_mesh,
    out_type=jax.ShapeDtypeStruct((EMBED_DIM, BATCH), jnp.float32),
    scratch_types=[
        pltpu.VMEM((_SPP,), jnp.int32),
        pltpu.VMEM((_SPP + 16,), jnp.int32),
        pltpu.VMEM((_SPP + 16,), jnp.int32),
        pltpu.VMEM((2 * _B, _DPW, 128), jnp.float32),
        pltpu.VMEM((_B, 48), jnp.float32),
        pltpu.VMEM((_DPW, _SPP + 16), jnp.float32),
        pltpu.SemaphoreType.DMA,
        pltpu.SemaphoreType.DMA,
    ],
)
def _gather_kernel(
    idx_hbm, tab_hbm, out_hbm, idx_v, voff_v, vlane_v, slab_v, spill_v,
    out_v, sem_a, sem_b
):
    wid = lax.axis_index("s") * _NC + lax.axis_index("c")
    pair = wid >> 1
    dbase = pl.multiple_of((wid & 1) * _DPW, _DPW)
    sbase = pair * _SPP
    pltpu.sync_copy(idx_hbm.at[pl.ds(sbase, _SPP)], idx_v)

    def col_body(v, carry):
        c16 = idx_v[pl.ds(v * 16, 16)]
        voff_v[pl.ds(v * 16, 16)] = (c16 >> 7) << 7
        vlane_v[pl.ds(v * 16, 16)] = c16 & 127
        return carry

    lax.fori_loop(0, _SPP // 16, col_body, 0)

    lane_iota = lax.iota(jnp.int32, 16)

    def fire(b, pbase, sem):
        voff16 = voff_v[pl.ds(b * _B, 16)]
        for j in range(_B):
            c128 = pl.multiple_of(voff16[j], 128)
            pltpu.async_copy(
                tab_hbm.at[pl.ds(dbase, _DPW), pl.ds(c128, 128)],
                slab_v.at[pbase + j],
                sem,
            )

    def drain(pbase, sem):
        for j in range(_B):
            pltpu.make_async_copy(
                tab_hbm.at[pl.ds(dbase, _DPW), pl.ds(0, 128)],
                slab_v.at[pbase + j],
                sem,
            ).wait()

    def extract(b, pbase):
        vlan16 = vlane_v[pl.ds(b * _B, 16)]
        g_out = pl.multiple_of((b >> 1) << 4, 16)
        pmasks, shifts, g16s = [], [], []
        for j in range(_B):
            pos = (b & 1) * _B + j
            lane = vlan16[j]
            pmasks.append(lane_iota == jnp.broadcast_to(pos, (16,)))
            g16s.append(pl.multiple_of((lane >> 4) << 4, 16))
            shifts.append(16 + (lane & 15) - pos)
        for d in range(_DPW):
            acc = out_v[d, pl.ds(g_out, 16)]
            for j in range(_B):
                vec = slab_v[pbase + j, d, pl.ds(g16s[j], 16)]
                spill_v[j, pl.ds(16, 16)] = vec
                moved = spill_v[j, pl.ds(shifts[j], 16)]
                acc = jnp.where(pmasks[j], moved, acc)
            out_v[d, pl.ds(g_out, 16)] = acc

    fire(0, 0, sem_a)

    def body(g, carry):
        fire(2 * g + 1, _B, sem_b)
        drain(0, sem_a)
        extract(2 * g, 0)
        fire(2 * g + 2, 0, sem_a)
        drain(_B, sem_b)
        extract(2 * g + 1, _B)
        return carry

    lax.fori_loop(0, _NBURST // 2 - 1, body, 0)

    fire(_NBURST - 1, _B, sem_b)
    drain(0, sem_a)
    extract(_NBURST - 2, 0)
    drain(_B, sem_b)
    extract(_NBURST - 1, _B)

    pltpu.sync_copy(
        out_v.at[:, pl.ds(0, _SPP)],
        out_hbm.at[pl.ds(dbase, _DPW), pl.ds(sbase, _SPP)],
    )


def kernel(class_labels, embedding_table):
    idx = class_labels.astype(jnp.int32)
    tab_t = jnp.swapaxes(embedding_table, 0, 1)
    out_t = _gather_kernel(idx, tab_t)
    return jnp.swapaxes(out_t, 0, 1).reshape(BATCH, 1, EMBED_DIM)
